# baseline (speedup 1.0000x reference)
import functools

import jax
import jax.numpy as jnp
from jax import lax
from jax.experimental import pallas as pl
from jax.experimental.pallas import tpu as pltpu
from jax.experimental.pallas import tpu_sc as plsc

B, L, C, F, V, D = 1024, 50, 26, 13, 100000, 32
N = B * L
OUT_C = C + F
VP = 100096
VB = VP // 128
NB = N // 128
KCH = 8
RCH = NB // KCH

_info = plsc.get_sparse_core_info()
NC, NS = _info.num_cores, _info.num_subcores
NW = NC * NS


def _make_sc_kernel():
    mesh = plsc.VectorSubcoreMesh(core_axis_name="c", subcore_axis_name="s")

    @functools.partial(
        pl.kernel,
        mesh=mesh,
        compiler_params=pltpu.CompilerParams(
            use_tc_tiling_on_sc=False, needs_layout_passes=False),
        out_type=jax.ShapeDtypeStruct((OUT_C, 4, NB, 8, 128), jnp.float32),
        scratch_types=[
            pltpu.VMEM((VB, 128), jnp.float32),
            pltpu.VMEM((RCH, 128), jnp.int32),
            pltpu.VMEM((RCH, 128), jnp.float32),
            pltpu.VMEM((RCH, 128), jnp.float32),
            pltpu.VMEM((16,), jnp.float32),
            pltpu.VMEM((16,), jnp.float32),
        ],
    )
    def sc_kernel(cat_hbm, cont_hbm, tab_hbm, w_hbm, b_hbm, out_hbm,
                  src_v, idx_v, x_v, o_v, wr_v, br_v):
        d = lax.axis_index("s") * NC + lax.axis_index("c")
        db = d // 8
        dm = d % 8

        pltpu.sync_copy(w_hbm.at[d], wr_v)
        pltpu.sync_copy(b_hbm.at[d], br_v)

        def cat_task(c, carry):
            pltpu.sync_copy(tab_hbm.at[c, db, :, dm, :], src_v)

            def chunk(k, cc):
                pltpu.sync_copy(cat_hbm.at[c, pl.ds(k * RCH, RCH), :], idx_v)

                def row(r, rc):
                    for h in range(8):
                        iv = idx_v[r, pl.ds(h * 16, 16)]
                        vb = jax.lax.shift_right_logical(iv, 7)
                        vm = jax.lax.bitwise_and(iv, 127)
                        o_v[r, pl.ds(h * 16, 16)] = plsc.load_gather(
                            src_v, [vb, vm])
                    return rc
                lax.fori_loop(0, RCH, row, 0, unroll=5)
                pltpu.sync_copy(
                    o_v, out_hbm.at[c, db, pl.ds(k * RCH, RCH), dm, :])
                return cc
            lax.fori_loop(0, KCH, chunk, 0)
            return carry
        lax.fori_loop(0, C, cat_task, 0)

        wrow = wr_v[...]
        brow = br_v[...]
        for f in range(F):
            ws = jnp.full((16,), wrow[f], dtype=jnp.float32)
            bs = jnp.full((16,), brow[f], dtype=jnp.float32)

            def chunk_f(k, cc, f=f, ws=ws, bs=bs):
                pltpu.sync_copy(cont_hbm.at[f, pl.ds(k * RCH, RCH), :], x_v)

                def row(r, rc):
                    for h in range(8):
                        xv = x_v[r, pl.ds(h * 16, 16)]
                        o_v[r, pl.ds(h * 16, 16)] = xv * ws + bs
                    return rc
                lax.fori_loop(0, RCH, row, 0, unroll=5)
                pltpu.sync_copy(
                    o_v, out_hbm.at[C + f, db, pl.ds(k * RCH, RCH), dm, :])
                return cc
            lax.fori_loop(0, KCH, chunk_f, 0)

    return sc_kernel


_sc_kernel = _make_sc_kernel()


def kernel(cat, cont, emb_tables, cont_W, cont_b):
    catT = cat.reshape(N, C).T.reshape(C, NB, 128)
    contT = cont.reshape(N, F).T.reshape(F, NB, 128)
    tabT = emb_tables.transpose(0, 2, 1)
    tabP = jnp.pad(tabT, ((0, 0), (0, 0), (0, VP - V)))
    tab5 = tabP.reshape(C, 4, 8, VB, 128).transpose(0, 1, 3, 2, 4)
    wT = jnp.zeros((32, 16), jnp.float32).at[:, :F].set(cont_W.T)
    bT = jnp.zeros((32, 16), jnp.float32).at[:, :F].set(cont_b.T)
    out5 = _sc_kernel(catT, contT, tab5, wT, bT)
    return out5.transpose(2, 4, 0, 1, 3).reshape(N, OUT_C, D)


# speedup vs baseline: 1.9706x; 1.9706x over previous
import functools

import jax
import jax.numpy as jnp
from jax import lax
from jax.experimental import pallas as pl
from jax.experimental.pallas import tpu as pltpu
from jax.experimental.pallas import tpu_sc as plsc

B, L, C, F, V, D = 1024, 50, 26, 13, 100000, 32
N = B * L
OUT_C = C + F
VP = 100096
VB = VP // 128
NB = N // 128
RCH = 25
KCH = NB // RCH
UTOT = F * KCH

_info = plsc.get_sparse_core_info()
NC, NS = _info.num_cores, _info.num_subcores
NW = NC * NS


def _make_sc_kernel():
    mesh = plsc.VectorSubcoreMesh(core_axis_name="c", subcore_axis_name="s")

    @functools.partial(
        pl.kernel,
        mesh=mesh,
        compiler_params=pltpu.CompilerParams(
            use_tc_tiling_on_sc=False, needs_layout_passes=False),
        out_type=jax.ShapeDtypeStruct((OUT_C, 4, NB, 8, 128), jnp.float32),
        scratch_types=[
            pltpu.VMEM((VB, 128), jnp.float32),
            pltpu.VMEM((RCH, 128), jnp.int32),
            pltpu.VMEM((RCH, 128), jnp.int32),
            pltpu.VMEM((RCH, 128), jnp.float32),
            pltpu.VMEM((RCH, 128), jnp.float32),
            pltpu.VMEM((RCH, 128), jnp.float32),
            pltpu.VMEM((RCH, 128), jnp.float32),
            pltpu.VMEM((RCH, 128), jnp.float32),
            pltpu.VMEM((RCH, 128), jnp.float32),
            pltpu.VMEM((16, 16), jnp.float32),
            pltpu.VMEM((16, 16), jnp.float32),
            pltpu.VMEM((16,), jnp.float32),
            pltpu.VMEM((16,), jnp.float32),
            pltpu.SemaphoreType.DMA,
            pltpu.SemaphoreType.DMA,
            pltpu.SemaphoreType.DMA,
            pltpu.SemaphoreType.DMA,
            pltpu.SemaphoreType.DMA,
            pltpu.SemaphoreType.DMA,
            pltpu.SemaphoreType.DMA,
            pltpu.SemaphoreType.DMA,
            pltpu.SemaphoreType.DMA,
        ],
    )
    def sc_kernel(cat_hbm, cont_hbm, tab_hbm, w_hbm, b_hbm, out_hbm,
                  src_v, i0_v, i1_v, x0_v, x1_v, oa0_v, oa1_v, of0_v, of1_v,
                  wsp_v, bsp_v, wr_v, br_v,
                  ss, si0, si1, sx0, sx1, sa0, sa1, sf0, sf1):
        d = lax.axis_index("s") * NC + lax.axis_index("c")
        db = d // 8
        dm = d % 8

        pltpu.sync_copy(w_hbm.at[d], wr_v)
        pltpu.sync_copy(b_hbm.at[d], br_v)
        wrow = wr_v[...]
        brow = br_v[...]
        for f in range(F):
            wsp_v[f, :] = jnp.full((16,), wrow[f], dtype=jnp.float32)
            bsp_v[f, :] = jnp.full((16,), brow[f], dtype=jnp.float32)

        def idx_copy(c, k, buf, sem):
            return pltpu.make_async_copy(
                cat_hbm.at[c, pl.ds(k * RCH, RCH), :], buf, sem)

        def x_copy(u, buf, sem):
            return pltpu.make_async_copy(
                cont_hbm.at[u // KCH, pl.ds((u % KCH) * RCH, RCH), :],
                buf, sem)

        def out_copy(j, k, buf, sem):
            return pltpu.make_async_copy(
                buf, out_hbm.at[j, db, pl.ds(k * RCH, RCH), dm, :], sem)

        def lookup_chunk(ib, ob):
            def row(r, rc):
                for h in range(8):
                    iv = ib[r, pl.ds(h * 16, 16)]
                    vb = jax.lax.shift_right_logical(iv, 7)
                    vm = jax.lax.bitwise_and(iv, 127)
                    ob[r, pl.ds(h * 16, 16)] = plsc.load_gather(
                        src_v, [vb, vm])
                return rc
            lax.fori_loop(0, RCH, row, 0)

        def lin_unit(u, xb, ob):
            ws = wsp_v[u // KCH, :]
            bs = bsp_v[u // KCH, :]

            def row(r, rc):
                for h in range(8):
                    xv = xb[r, pl.ds(h * 16, 16)]
                    ob[r, pl.ds(h * 16, 16)] = xv * ws + bs
                return rc
            lax.fori_loop(0, RCH, row, 0)

        x_copy(0, x0_v, sx0).start()
        x_copy(1, x1_v, sx1).start()

        def cat_task(c, carry):
            src_cp = pltpu.make_async_copy(
                tab_hbm.at[c, db, :, dm, :], src_v, ss)
            src_cp.start()

            def cont_ss(js, cc):
                u0 = js * 2
                x_copy(u0, x0_v, sx0).wait()

                @pl.when(js > 0)
                def _():
                    out_copy(C, 0, of0_v, sf0).wait()
                lin_unit(u0, x0_v, of0_v)
                out_copy(C + u0 // KCH, u0 % KCH, of0_v, sf0).start()

                @pl.when(u0 + 2 < UTOT)
                def _():
                    x_copy(u0 + 2, x0_v, sx0).start()

                x_copy(u0 + 1, x1_v, sx1).wait()

                @pl.when(js > 0)
                def _():
                    out_copy(C, 0, of1_v, sf1).wait()
                lin_unit(u0 + 1, x1_v, of1_v)
                out_copy(C + (u0 + 1) // KCH, (u0 + 1) % KCH, of1_v,
                         sf1).start()

                @pl.when(u0 + 3 < UTOT)
                def _():
                    x_copy(u0 + 3, x1_v, sx1).start()
                return cc
            lax.fori_loop(c * 4, c * 4 + 4, cont_ss, 0)

            src_cp.wait()

            idx_copy(c, 0, i0_v, si0).start()

            def superstep(ks, cc):
                k0 = ks * 2
                idx_copy(c, k0, i0_v, si0).wait()
                idx_copy(c, k0 + 1, i1_v, si1).start()

                @pl.when(ks > 0)
                def _():
                    out_copy(c, 0, oa0_v, sa0).wait()
                lookup_chunk(i0_v, oa0_v)
                out_copy(c, k0, oa0_v, sa0).start()

                idx_copy(c, k0 + 1, i1_v, si1).wait()

                @pl.when(ks < KCH // 2 - 1)
                def _():
                    idx_copy(c, k0 + 2, i0_v, si0).start()

                @pl.when(ks > 0)
                def _():
                    out_copy(c, 0, oa1_v, sa1).wait()
                lookup_chunk(i1_v, oa1_v)
                out_copy(c, k0 + 1, oa1_v, sa1).start()
                return cc
            lax.fori_loop(0, KCH // 2, superstep, 0)
            out_copy(c, KCH - 2, oa0_v, sa0).wait()
            out_copy(c, KCH - 1, oa1_v, sa1).wait()
            return carry
        lax.fori_loop(0, C, cat_task, 0)

        out_copy(C, 0, of0_v, sf0).wait()
        out_copy(C, 0, of1_v, sf1).wait()

    return sc_kernel


_sc_kernel = _make_sc_kernel()


def kernel(cat, cont, emb_tables, cont_W, cont_b):
    catT = cat.reshape(N, C).T.reshape(C, NB, 128)
    contT = cont.reshape(N, F).T.reshape(F, NB, 128)
    tabT = emb_tables.transpose(0, 2, 1)
    tabP = jnp.pad(tabT, ((0, 0), (0, 0), (0, VP - V)))
    tab5 = tabP.reshape(C, 4, 8, VB, 128).transpose(0, 1, 3, 2, 4)
    wT = jnp.zeros((32, 16), jnp.float32).at[:, :F].set(cont_W.T)
    bT = jnp.zeros((32, 16), jnp.float32).at[:, :F].set(cont_b.T)
    out5 = _sc_kernel(catT, contT, tab5, wT, bT)
    return out5.transpose(2, 4, 0, 1, 3).reshape(N, OUT_C, D)
